# baseline (speedup 1.0000x reference)
import functools

import jax
import jax.numpy as jnp
from jax import lax
from jax.experimental import pallas as pl
from jax.experimental.pallas import tpu as pltpu
from jax.experimental.pallas import tpu_sc as plsc

H = 1024
I = 2816
E = 8
K = 2
B, S = 2, 2048
T = B * S
A = K * T

TB = 512
NTB = T // TB

TILE = 256
NT = A // TILE + E
PAD = NT * TILE
NJ = 2
IC = I // NJ


def _router_body(x_ref, r_ref, i1_ref, i2_ref, w1_ref, w2_ref, r0_ref, r1_ref,
                 cnt_ref, stats_ref, carry, psum, zsq):
    bi = pl.program_id(0)

    @pl.when(bi == 0)
    def _init():
        carry[...] = jnp.zeros_like(carry)
        psum[...] = jnp.zeros_like(psum)
        zsq[0, 0] = 0.0

    x = x_ref[...]
    logits = jnp.dot(x, r_ref[...], preferred_element_type=jnp.float32)

    lanes = jax.lax.broadcasted_iota(jnp.int32, (TB, E), 1)
    m1 = jnp.max(logits, axis=-1, keepdims=True)
    i1 = jnp.min(jnp.where(logits == m1, lanes, E), axis=-1, keepdims=True)
    oh1 = (lanes == i1).astype(jnp.float32)
    l2 = jnp.where(lanes == i1, jnp.float32(-1e30), logits)
    m2 = jnp.max(l2, axis=-1, keepdims=True)
    i2 = jnp.min(jnp.where(l2 == m2, lanes, E), axis=-1, keepdims=True)
    oh2 = (lanes == i2).astype(jnp.float32)

    e2 = jnp.exp(m2 - m1)
    w1 = 1.0 / (1.0 + e2)
    w2 = e2 / (1.0 + e2)

    ex = jnp.exp(logits - m1)
    sex = jnp.sum(ex, axis=-1, keepdims=True)
    probs = ex / sex
    z = m1 + jnp.log(sex)

    rows = jax.lax.broadcasted_iota(jnp.int32, (TB, TB), 0)
    cols = jax.lax.broadcasted_iota(jnp.int32, (TB, TB), 1)
    tril = (cols < rows).astype(jnp.float32)
    ohb = oh1 + oh2
    base = jnp.dot(tril, ohb, preferred_element_type=jnp.float32) + carry[...]
    r0 = jnp.sum(base * oh1, axis=-1, keepdims=True)
    r1 = jnp.sum(base * oh2, axis=-1, keepdims=True)

    carry[...] = carry[...] + jnp.sum(ohb, axis=0, keepdims=True)
    psum[...] = psum[...] + jnp.sum(probs, axis=0, keepdims=True)
    zsq[0, 0] = zsq[0, 0] + jnp.sum(z * z)

    i1_ref[...] = i1
    i2_ref[...] = i2
    w1_ref[...] = w1
    w2_ref[...] = w2
    r0_ref[...] = r0
    r1_ref[...] = r1

    @pl.when(bi == NTB - 1)
    def _fin():
        cnt = carry[...]
        cnt_ref[...] = cnt
        af = cnt / jnp.float32(A)
        ent = -jnp.sum(af * jnp.log(af + 1e-6))
        pm = psum[...] / jnp.float32(T)
        lb = jnp.float32(E) * jnp.sum(af * jnp.float32(K) * pm)
        stats_ref[0, 0] = ent
        stats_ref[0, 1] = lb
        stats_ref[0, 2] = zsq[0, 0] / jnp.float32(T)


def _router_call(x_flat, router):
    f32 = jnp.float32
    outs = (
        jax.ShapeDtypeStruct((T, 1), jnp.int32),
        jax.ShapeDtypeStruct((T, 1), jnp.int32),
        jax.ShapeDtypeStruct((T, 1), f32),
        jax.ShapeDtypeStruct((T, 1), f32),
        jax.ShapeDtypeStruct((T, 1), f32),
        jax.ShapeDtypeStruct((T, 1), f32),
        jax.ShapeDtypeStruct((1, E), f32),
        jax.ShapeDtypeStruct((1, 8), f32),
    )
    col = pl.BlockSpec((TB, 1), lambda i: (i, 0))
    return pl.pallas_call(
        _router_body,
        grid=(NTB,),
        in_specs=[
            pl.BlockSpec((TB, H), lambda i: (i, 0)),
            pl.BlockSpec((H, E), lambda i: (0, 0)),
        ],
        out_specs=(col, col, col, col, col, col,
                   pl.BlockSpec((1, E), lambda i: (0, 0)),
                   pl.BlockSpec((1, 8), lambda i: (0, 0),
                                memory_space=pltpu.SMEM)),
        out_shape=outs,
        scratch_shapes=[
            pltpu.VMEM((1, E), f32),
            pltpu.VMEM((1, E), f32),
            pltpu.SMEM((1, 1), f32),
        ],
    )(x_flat, router)


def _mlp_body(te_ref, act_ref, xs_ref, wg_ref, wu_ref, wd_ref, out_ref):
    i = pl.program_id(0)
    j = pl.program_id(1)

    @pl.when(act_ref[i] == 1)
    def _():
        xs = xs_ref[...]
        g = jnp.dot(xs, wg_ref[0], preferred_element_type=jnp.float32)
        u = jnp.dot(xs, wu_ref[0], preferred_element_type=jnp.float32)
        hh = g * jax.nn.sigmoid(g) * u
        y = jnp.dot(hh, wd_ref[0], preferred_element_type=jnp.float32)

        @pl.when(j == 0)
        def _():
            out_ref[...] = y

        @pl.when(j != 0)
        def _():
            out_ref[...] = out_ref[...] + y


def _snake(i, j):
    return jnp.where(i % 2 == 1, NJ - 1 - j, j)


def _mlp_call(te, act, xs, w_up_gate, w_down):
    grid_spec = pltpu.PrefetchScalarGridSpec(
        num_scalar_prefetch=2,
        grid=(NT, NJ),
        in_specs=[
            pl.BlockSpec((TILE, H), lambda i, j, te, act: (i, 0)),
            pl.BlockSpec((1, H, IC),
                         lambda i, j, te, act: (te[i], 0, _snake(i, j))),
            pl.BlockSpec((1, H, IC),
                         lambda i, j, te, act: (te[i], 0, NJ + _snake(i, j))),
            pl.BlockSpec((1, IC, H),
                         lambda i, j, te, act: (te[i], _snake(i, j), 0)),
        ],
        out_specs=pl.BlockSpec((TILE, H), lambda i, j, te, act: (i, 0)),
    )
    return pl.pallas_call(
        _mlp_body,
        grid_spec=grid_spec,
        out_shape=jax.ShapeDtypeStruct((PAD, H), jnp.float32),
    )(te, act, xs, w_up_gate, w_up_gate, w_down)


_NC, _NS = 2, 16
_NW = _NC * _NS
_CH = 32


def _sc_gather(table, idx, n_rows):
    b_per_w = n_rows // _NW
    nch = b_per_w // _CH
    mesh = plsc.VectorSubcoreMesh(core_axis_name="c", subcore_axis_name="s")

    @functools.partial(
        pl.kernel, mesh=mesh,
        out_type=jax.ShapeDtypeStruct((n_rows, H), jnp.float32),
        scratch_types=[
            pltpu.VMEM((b_per_w,), jnp.int32),
            pltpu.VMEM((_CH, H), jnp.float32),
            pltpu.VMEM((_CH, H), jnp.float32),
            pltpu.SemaphoreType.DMA,
            pltpu.SemaphoreType.DMA,
            pltpu.SemaphoreType.DMA,
            pltpu.SemaphoreType.DMA,
        ],
    )
    def k(table_hbm, idx_hbm, out_hbm, idx_v, buf0, buf1, ga, gb, sa, sb):
        wid = lax.axis_index("s") * _NC + lax.axis_index("c")
        base = wid * b_per_w
        pltpu.sync_copy(idx_hbm.at[pl.ds(base, b_per_w)], idx_v)
        bufs = (buf0, buf1)
        gsem = (ga, gb)
        ssem = (sa, sb)

        def gather(j):
            return pltpu.async_copy(
                table_hbm.at[idx_v.at[pl.ds(j * _CH, _CH)]], bufs[j % 2],
                gsem[j % 2])

        def store(j):
            return pltpu.async_copy(
                bufs[j % 2], out_hbm.at[pl.ds(base + j * _CH, _CH)],
                ssem[j % 2])

        g = gather(0)
        st = {}
        for j in range(nch):
            if j + 1 < nch:
                if j >= 1:
                    st[j - 1].wait()
                g_next = gather(j + 1)
            g.wait()
            st[j] = store(j)
            if j + 1 < nch:
                g = g_next
        if nch >= 2:
            st[nch - 2].wait()
        st[nch - 1].wait()

    return k(table, idx)


def _comb_body(g0_ref, g1_ref, w1_ref, w2_ref, o_ref):
    o_ref[...] = w1_ref[...] * g0_ref[...] + w2_ref[...] * g1_ref[...]


def _comb_call(gall, w1, w2):
    row = pl.BlockSpec((TB, H), lambda i: (i, 0))
    row1 = pl.BlockSpec((TB, H), lambda i: (i + NTB, 0))
    col = pl.BlockSpec((TB, 1), lambda i: (i, 0))
    return pl.pallas_call(
        _comb_body,
        grid=(NTB,),
        in_specs=[row, row1, col, col],
        out_specs=row,
        out_shape=jax.ShapeDtypeStruct((T, H), jnp.float32),
    )(gall, gall, w1, w2)


def kernel(x, router, w_up_gate, w_down):
    x_flat = x.reshape(T, H)
    i1, i2, w1, w2, r0, r1, cnt, stats = _router_call(x_flat, router)

    cnt_v = cnt[0]
    cnt_i = cnt_v.astype(jnp.int32)
    tiles_e = (cnt_i + TILE - 1) // TILE
    seg = jnp.concatenate([jnp.zeros(1, jnp.int32),
                           jnp.cumsum(tiles_e * TILE)[:-1]])
    pos0 = seg[i1[:, 0]] + r0[:, 0].astype(jnp.int32)
    pos1 = seg[i2[:, 0]] + r1[:, 0].astype(jnp.int32)
    tok = jnp.arange(T, dtype=jnp.int32)
    src = jnp.zeros(PAD, jnp.int32).at[pos0].set(tok).at[pos1].set(tok)

    cumt = jnp.cumsum(tiles_e)
    ti = jnp.arange(NT, dtype=jnp.int32)
    te = jnp.minimum(jnp.sum((ti[:, None] >= cumt[None, :]).astype(jnp.int32),
                             axis=1), E - 1).astype(jnp.int32)
    act = (ti < cumt[-1]).astype(jnp.int32)

    xs = _sc_gather(x_flat, src, PAD)
    ys = _mlp_call(te, act, xs, w_up_gate, w_down)
    gall = _sc_gather(ys, jnp.concatenate([pos0, pos1]), A)
    routed = _comb_call(gall, w1, w2).reshape(B, S, H)

    return (routed, cnt_v, stats[0, 0], stats[0, 1], stats[0, 2])


# speedup vs baseline: 1.3255x; 1.3255x over previous
import functools

import jax
import jax.numpy as jnp
from jax import lax
from jax.experimental import pallas as pl
from jax.experimental.pallas import tpu as pltpu
from jax.experimental.pallas import tpu_sc as plsc

H = 1024
I = 2816
E = 8
K = 2
B, S = 2, 2048
T = B * S
A = K * T

TB = 512
NTB = T // TB

TILE = 256
NT = A // TILE + E
PAD = NT * TILE
NJ = 2
IC = I // NJ


def _router_body(x_ref, r_ref, i1_ref, i2_ref, w1_ref, w2_ref, r0_ref, r1_ref,
                 cnt_ref, stats_ref, carry, psum, zsq):
    bi = pl.program_id(0)

    @pl.when(bi == 0)
    def _init():
        carry[...] = jnp.zeros_like(carry)
        psum[...] = jnp.zeros_like(psum)
        zsq[0, 0] = 0.0

    x = x_ref[...]
    logits = jnp.dot(x, r_ref[...], preferred_element_type=jnp.float32)

    lanes = jax.lax.broadcasted_iota(jnp.int32, (TB, E), 1)
    m1 = jnp.max(logits, axis=-1, keepdims=True)
    i1 = jnp.min(jnp.where(logits == m1, lanes, E), axis=-1, keepdims=True)
    oh1 = (lanes == i1).astype(jnp.float32)
    l2 = jnp.where(lanes == i1, jnp.float32(-1e30), logits)
    m2 = jnp.max(l2, axis=-1, keepdims=True)
    i2 = jnp.min(jnp.where(l2 == m2, lanes, E), axis=-1, keepdims=True)
    oh2 = (lanes == i2).astype(jnp.float32)

    e2 = jnp.exp(m2 - m1)
    w1 = 1.0 / (1.0 + e2)
    w2 = e2 / (1.0 + e2)

    ex = jnp.exp(logits - m1)
    sex = jnp.sum(ex, axis=-1, keepdims=True)
    probs = ex / sex
    z = m1 + jnp.log(sex)

    rows = jax.lax.broadcasted_iota(jnp.int32, (TB, TB), 0)
    cols = jax.lax.broadcasted_iota(jnp.int32, (TB, TB), 1)
    tril = (cols < rows).astype(jnp.float32)
    ohb = oh1 + oh2
    base = jnp.dot(tril, ohb, preferred_element_type=jnp.float32) + carry[...]
    r0 = jnp.sum(base * oh1, axis=-1, keepdims=True)
    r1 = jnp.sum(base * oh2, axis=-1, keepdims=True)

    carry[...] = carry[...] + jnp.sum(ohb, axis=0, keepdims=True)
    psum[...] = psum[...] + jnp.sum(probs, axis=0, keepdims=True)
    zsq[0, 0] = zsq[0, 0] + jnp.sum(z * z)

    i1_ref[...] = i1
    i2_ref[...] = i2
    w1_ref[...] = w1
    w2_ref[...] = w2
    r0_ref[...] = r0
    r1_ref[...] = r1

    @pl.when(bi == NTB - 1)
    def _fin():
        cnt = carry[...]
        cnt_ref[...] = cnt
        af = cnt / jnp.float32(A)
        ent = -jnp.sum(af * jnp.log(af + 1e-6))
        pm = psum[...] / jnp.float32(T)
        lb = jnp.float32(E) * jnp.sum(af * jnp.float32(K) * pm)
        stats_ref[0, 0] = ent
        stats_ref[0, 1] = lb
        stats_ref[0, 2] = zsq[0, 0] / jnp.float32(T)


def _router_call(x_flat, router):
    f32 = jnp.float32
    outs = (
        jax.ShapeDtypeStruct((T, 1), jnp.int32),
        jax.ShapeDtypeStruct((T, 1), jnp.int32),
        jax.ShapeDtypeStruct((T, 1), f32),
        jax.ShapeDtypeStruct((T, 1), f32),
        jax.ShapeDtypeStruct((T, 1), f32),
        jax.ShapeDtypeStruct((T, 1), f32),
        jax.ShapeDtypeStruct((1, E), f32),
        jax.ShapeDtypeStruct((1, 8), f32),
    )
    col = pl.BlockSpec((TB, 1), lambda i: (i, 0))
    return pl.pallas_call(
        _router_body,
        grid=(NTB,),
        in_specs=[
            pl.BlockSpec((TB, H), lambda i: (i, 0)),
            pl.BlockSpec((H, E), lambda i: (0, 0)),
        ],
        out_specs=(col, col, col, col, col, col,
                   pl.BlockSpec((1, E), lambda i: (0, 0)),
                   pl.BlockSpec((1, 8), lambda i: (0, 0),
                                memory_space=pltpu.SMEM)),
        out_shape=outs,
        scratch_shapes=[
            pltpu.VMEM((1, E), f32),
            pltpu.VMEM((1, E), f32),
            pltpu.SMEM((1, 1), f32),
        ],
    )(x_flat, router)


def _mlp_body(te_ref, act_ref, xs_ref, wg_ref, wu_ref, wd_ref, out_ref):
    i = pl.program_id(0)
    j = pl.program_id(1)

    @pl.when(act_ref[i] == 1)
    def _():
        xs = xs_ref[...]
        g = jnp.dot(xs, wg_ref[0], preferred_element_type=jnp.float32)
        u = jnp.dot(xs, wu_ref[0], preferred_element_type=jnp.float32)
        hh = g * jax.nn.sigmoid(g) * u
        y = jnp.dot(hh, wd_ref[0], preferred_element_type=jnp.float32)

        @pl.when(j == 0)
        def _():
            out_ref[...] = y

        @pl.when(j != 0)
        def _():
            out_ref[...] = out_ref[...] + y


def _snake(i, j):
    return jnp.where(i % 2 == 1, NJ - 1 - j, j)


def _mlp_call(te, act, xs, w_up_gate, w_down):
    grid_spec = pltpu.PrefetchScalarGridSpec(
        num_scalar_prefetch=2,
        grid=(NT, NJ),
        in_specs=[
            pl.BlockSpec((TILE, H), lambda i, j, te, act: (i, 0)),
            pl.BlockSpec((1, H, IC),
                         lambda i, j, te, act: (te[i], 0, _snake(i, j))),
            pl.BlockSpec((1, H, IC),
                         lambda i, j, te, act: (te[i], 0, NJ + _snake(i, j))),
            pl.BlockSpec((1, IC, H),
                         lambda i, j, te, act: (te[i], _snake(i, j), 0)),
        ],
        out_specs=pl.BlockSpec((TILE, H), lambda i, j, te, act: (i, 0)),
    )
    return pl.pallas_call(
        _mlp_body,
        grid_spec=grid_spec,
        out_shape=jax.ShapeDtypeStruct((PAD, H), jnp.float32),
    )(te, act, xs, w_up_gate, w_up_gate, w_down)


_NC, _NS = 2, 16
_NW = _NC * _NS
_CH = 32


def _sc_gather(table, idx, n_rows):
    b_per_w = n_rows // _NW
    nch = b_per_w // _CH
    mesh = plsc.VectorSubcoreMesh(core_axis_name="c", subcore_axis_name="s")

    @functools.partial(
        pl.kernel, mesh=mesh,
        out_type=jax.ShapeDtypeStruct((n_rows, H), jnp.float32),
        scratch_types=[
            pltpu.VMEM((b_per_w,), jnp.int32),
            pltpu.VMEM((_CH, H), jnp.float32),
            pltpu.VMEM((_CH, H), jnp.float32),
            pltpu.SemaphoreType.DMA,
            pltpu.SemaphoreType.DMA,
            pltpu.SemaphoreType.DMA,
            pltpu.SemaphoreType.DMA,
        ],
    )
    def k(table_hbm, idx_hbm, out_hbm, idx_v, buf0, buf1, ga, gb, sa, sb):
        wid = lax.axis_index("s") * _NC + lax.axis_index("c")
        base = wid * b_per_w
        pltpu.sync_copy(idx_hbm.at[pl.ds(base, b_per_w)], idx_v)
        bufs = (buf0, buf1)
        gsem = (ga, gb)
        ssem = (sa, sb)

        def gather(j):
            return pltpu.async_copy(
                table_hbm.at[idx_v.at[pl.ds(j * _CH, _CH)]], bufs[j % 2],
                gsem[j % 2])

        def store(j):
            return pltpu.async_copy(
                bufs[j % 2], out_hbm.at[pl.ds(base + j * _CH, _CH)],
                ssem[j % 2])

        g = gather(0)
        st = {}
        for j in range(nch):
            if j + 1 < nch:
                if j >= 1:
                    st[j - 1].wait()
                g_next = gather(j + 1)
            g.wait()
            st[j] = store(j)
            if j + 1 < nch:
                g = g_next
        if nch >= 2:
            st[nch - 2].wait()
        st[nch - 1].wait()

    return k(table, idx)


_TPW = T // _NW
_NC2 = _TPW // _CH


def _sc_dispatch(x_flat, posd):
    mesh = plsc.VectorSubcoreMesh(core_axis_name="c", subcore_axis_name="s")

    @functools.partial(
        pl.kernel, mesh=mesh,
        out_type=jax.ShapeDtypeStruct((PAD, H), jnp.float32),
        scratch_types=[
            pltpu.VMEM((2, _NC2, _CH), jnp.int32),
            pltpu.VMEM((_CH, H), jnp.float32),
            pltpu.VMEM((_CH, H), jnp.float32),
            pltpu.SemaphoreType.DMA,
            pltpu.SemaphoreType.DMA,
            pltpu.SemaphoreType.DMA,
            pltpu.SemaphoreType.DMA,
        ],
    )
    def k(x_hbm, pos_hbm, xs_hbm, idx_v, buf0, buf1, ra, rb, sa, sb):
        wid = lax.axis_index("s") * _NC + lax.axis_index("c")
        base = wid * _TPW
        pltpu.sync_copy(pos_hbm.at[wid], idx_v)
        bufs = (buf0, buf1)
        rsem = (ra, rb)
        ssem = (sa, sb)

        def read(c):
            return pltpu.async_copy(
                x_hbm.at[pl.ds(base + c * _CH, _CH)], bufs[c % 2], rsem[c % 2])

        def scat(c, slot):
            return pltpu.async_copy(
                bufs[c % 2], xs_hbm.at[idx_v.at[slot, c]], ssem[c % 2])

        rd = read(0)
        wr = {}
        for c in range(_NC2):
            if c + 1 < _NC2:
                if c >= 1:
                    wr[(c - 1, 0)].wait()
                    wr[(c - 1, 1)].wait()
                rd_next = read(c + 1)
            rd.wait()
            wr[(c, 0)] = scat(c, 0)
            wr[(c, 1)] = scat(c, 1)
            if c + 1 < _NC2:
                rd = rd_next
        for c in (_NC2 - 2, _NC2 - 1):
            wr[(c, 0)].wait()
            wr[(c, 1)].wait()

    return k(x_flat, posd)


def _comb_body(g0_ref, g1_ref, w1_ref, w2_ref, o_ref):
    o_ref[...] = w1_ref[...] * g0_ref[...] + w2_ref[...] * g1_ref[...]


def _comb_call(gall, w1, w2):
    row = pl.BlockSpec((TB, H), lambda i: (i, 0))
    row1 = pl.BlockSpec((TB, H), lambda i: (i + NTB, 0))
    col = pl.BlockSpec((TB, 1), lambda i: (i, 0))
    return pl.pallas_call(
        _comb_body,
        grid=(NTB,),
        in_specs=[row, row1, col, col],
        out_specs=row,
        out_shape=jax.ShapeDtypeStruct((T, H), jnp.float32),
    )(gall, gall, w1, w2)


def kernel(x, router, w_up_gate, w_down):
    x_flat = x.reshape(T, H)
    i1, i2, w1, w2, r0, r1, cnt, stats = _router_call(x_flat, router)

    cnt_v = cnt[0]
    cnt_i = cnt_v.astype(jnp.int32)
    tiles_e = (cnt_i + TILE - 1) // TILE
    seg = jnp.concatenate([jnp.zeros(1, jnp.int32),
                           jnp.cumsum(tiles_e * TILE)[:-1]])
    pos0 = seg[i1[:, 0]] + r0[:, 0].astype(jnp.int32)
    pos1 = seg[i2[:, 0]] + r1[:, 0].astype(jnp.int32)
    posd = jnp.stack([pos0.reshape(_NW, _NC2, _CH),
                      pos1.reshape(_NW, _NC2, _CH)], axis=1)

    cumt = jnp.cumsum(tiles_e)
    ti = jnp.arange(NT, dtype=jnp.int32)
    te = jnp.minimum(jnp.sum((ti[:, None] >= cumt[None, :]).astype(jnp.int32),
                             axis=1), E - 1).astype(jnp.int32)
    act = (ti < cumt[-1]).astype(jnp.int32)

    xs = _sc_dispatch(x_flat, posd)
    ys = _mlp_call(te, act, xs, w_up_gate, w_down)
    gall = _sc_gather(ys, jnp.concatenate([pos0, pos1]), A)
    routed = _comb_call(gall, w1, w2).reshape(B, S, H)

    return (routed, cnt_v, stats[0, 0], stats[0, 1], stats[0, 2])


# speedup vs baseline: 1.6623x; 1.2541x over previous
import functools

import jax
import jax.numpy as jnp
from jax import lax
from jax.experimental import pallas as pl
from jax.experimental.pallas import tpu as pltpu
from jax.experimental.pallas import tpu_sc as plsc

H = 1024
I = 2816
E = 8
K = 2
B, S = 2, 2048
T = B * S
A = K * T

TB = 512
NTB = T // TB

TILE = 512
NT = A // TILE + E
PAD = NT * TILE
NJ = 2
IC = I // NJ


def _router_body(x_ref, r_ref, i1_ref, i2_ref, w1_ref, w2_ref, r0_ref, r1_ref,
                 cnt_ref, stats_ref, carry, psum, zsq):
    bi = pl.program_id(0)

    @pl.when(bi == 0)
    def _init():
        carry[...] = jnp.zeros_like(carry)
        psum[...] = jnp.zeros_like(psum)
        zsq[0, 0] = 0.0

    x = x_ref[...]
    logits = jnp.dot(x, r_ref[...], preferred_element_type=jnp.float32)

    lanes = jax.lax.broadcasted_iota(jnp.int32, (TB, E), 1)
    m1 = jnp.max(logits, axis=-1, keepdims=True)
    i1 = jnp.min(jnp.where(logits == m1, lanes, E), axis=-1, keepdims=True)
    oh1 = (lanes == i1).astype(jnp.float32)
    l2 = jnp.where(lanes == i1, jnp.float32(-1e30), logits)
    m2 = jnp.max(l2, axis=-1, keepdims=True)
    i2 = jnp.min(jnp.where(l2 == m2, lanes, E), axis=-1, keepdims=True)
    oh2 = (lanes == i2).astype(jnp.float32)

    e2 = jnp.exp(m2 - m1)
    w1 = 1.0 / (1.0 + e2)
    w2 = e2 / (1.0 + e2)

    ex = jnp.exp(logits - m1)
    sex = jnp.sum(ex, axis=-1, keepdims=True)
    probs = ex / sex
    z = m1 + jnp.log(sex)

    rows = jax.lax.broadcasted_iota(jnp.int32, (TB, TB), 0)
    cols = jax.lax.broadcasted_iota(jnp.int32, (TB, TB), 1)
    tril = (cols < rows).astype(jnp.float32)
    ohb = oh1 + oh2
    base = jnp.dot(tril, ohb, preferred_element_type=jnp.float32) + carry[...]
    r0 = jnp.sum(base * oh1, axis=-1, keepdims=True)
    r1 = jnp.sum(base * oh2, axis=-1, keepdims=True)

    carry[...] = carry[...] + jnp.sum(ohb, axis=0, keepdims=True)
    psum[...] = psum[...] + jnp.sum(probs, axis=0, keepdims=True)
    zsq[0, 0] = zsq[0, 0] + jnp.sum(z * z)

    i1_ref[...] = i1
    i2_ref[...] = i2
    w1_ref[...] = w1
    w2_ref[...] = w2
    r0_ref[...] = r0
    r1_ref[...] = r1

    @pl.when(bi == NTB - 1)
    def _fin():
        cnt = carry[...]
        cnt_ref[...] = cnt
        af = cnt / jnp.float32(A)
        ent = -jnp.sum(af * jnp.log(af + 1e-6))
        pm = psum[...] / jnp.float32(T)
        lb = jnp.float32(E) * jnp.sum(af * jnp.float32(K) * pm)
        stats_ref[0, 0] = ent
        stats_ref[0, 1] = lb
        stats_ref[0, 2] = zsq[0, 0] / jnp.float32(T)


def _router_call(x_flat, router):
    f32 = jnp.float32
    outs = (
        jax.ShapeDtypeStruct((T, 1), jnp.int32),
        jax.ShapeDtypeStruct((T, 1), jnp.int32),
        jax.ShapeDtypeStruct((T, 1), f32),
        jax.ShapeDtypeStruct((T, 1), f32),
        jax.ShapeDtypeStruct((T, 1), f32),
        jax.ShapeDtypeStruct((T, 1), f32),
        jax.ShapeDtypeStruct((1, E), f32),
        jax.ShapeDtypeStruct((1, 8), f32),
    )
    col = pl.BlockSpec((TB, 1), lambda i: (i, 0))
    return pl.pallas_call(
        _router_body,
        grid=(NTB,),
        in_specs=[
            pl.BlockSpec((TB, H), lambda i: (i, 0)),
            pl.BlockSpec((H, E), lambda i: (0, 0)),
        ],
        out_specs=(col, col, col, col, col, col,
                   pl.BlockSpec((1, E), lambda i: (0, 0)),
                   pl.BlockSpec((1, 8), lambda i: (0, 0),
                                memory_space=pltpu.SMEM)),
        out_shape=outs,
        scratch_shapes=[
            pltpu.VMEM((1, E), f32),
            pltpu.VMEM((1, E), f32),
            pltpu.SMEM((1, 1), f32),
        ],
    )(x_flat, router)


def _mlp_body(te_ref, act_ref, xs_ref, wg_ref, wu_ref, wd_ref, out_ref):
    i = pl.program_id(0)
    j = pl.program_id(1)

    @pl.when(act_ref[i] == 1)
    def _():
        xs = xs_ref[...]
        g = jnp.dot(xs, wg_ref[0], preferred_element_type=jnp.float32)
        u = jnp.dot(xs, wu_ref[0], preferred_element_type=jnp.float32)
        hh = g * jax.nn.sigmoid(g) * u
        y = jnp.dot(hh, wd_ref[0], preferred_element_type=jnp.float32)

        @pl.when(j == 0)
        def _():
            out_ref[...] = y

        @pl.when(j != 0)
        def _():
            out_ref[...] = out_ref[...] + y


def _snake(i, j):
    return jnp.where(i % 2 == 1, NJ - 1 - j, j)


def _mlp_call(te, act, xs, w_up_gate, w_down):
    grid_spec = pltpu.PrefetchScalarGridSpec(
        num_scalar_prefetch=2,
        grid=(NT, NJ),
        in_specs=[
            pl.BlockSpec((TILE, H), lambda i, j, te, act: (i, 0)),
            pl.BlockSpec((1, H, IC),
                         lambda i, j, te, act: (te[i], 0, _snake(i, j))),
            pl.BlockSpec((1, H, IC),
                         lambda i, j, te, act: (te[i], 0, NJ + _snake(i, j))),
            pl.BlockSpec((1, IC, H),
                         lambda i, j, te, act: (te[i], _snake(i, j), 0)),
        ],
        out_specs=pl.BlockSpec((TILE, H), lambda i, j, te, act: (i, 0)),
    )
    return pl.pallas_call(
        _mlp_body,
        grid_spec=grid_spec,
        out_shape=jax.ShapeDtypeStruct((PAD, H), jnp.float32),
    )(te, act, xs, w_up_gate, w_up_gate, w_down)


_NC, _NS = 2, 16
_NW = _NC * _NS
_CH = 32


def _sc_gather(table, idx, n_rows):
    b_per_w = n_rows // _NW
    nch = b_per_w // _CH
    mesh = plsc.VectorSubcoreMesh(core_axis_name="c", subcore_axis_name="s")

    @functools.partial(
        pl.kernel, mesh=mesh,
        out_type=jax.ShapeDtypeStruct((n_rows, H), jnp.float32),
        scratch_types=[
            pltpu.VMEM((b_per_w,), jnp.int32),
            pltpu.VMEM((_CH, H), jnp.float32),
            pltpu.VMEM((_CH, H), jnp.float32),
            pltpu.SemaphoreType.DMA,
            pltpu.SemaphoreType.DMA,
            pltpu.SemaphoreType.DMA,
            pltpu.SemaphoreType.DMA,
        ],
    )
    def k(table_hbm, idx_hbm, out_hbm, idx_v, buf0, buf1, ga, gb, sa, sb):
        wid = lax.axis_index("s") * _NC + lax.axis_index("c")
        base = wid * b_per_w
        pltpu.sync_copy(idx_hbm.at[pl.ds(base, b_per_w)], idx_v)
        bufs = (buf0, buf1)
        gsem = (ga, gb)
        ssem = (sa, sb)

        def gather(j):
            return pltpu.async_copy(
                table_hbm.at[idx_v.at[pl.ds(j * _CH, _CH)]], bufs[j % 2],
                gsem[j % 2])

        def store(j):
            return pltpu.async_copy(
                bufs[j % 2], out_hbm.at[pl.ds(base + j * _CH, _CH)],
                ssem[j % 2])

        g = gather(0)
        st = {}
        for j in range(nch):
            if j + 1 < nch:
                if j >= 1:
                    st[j - 1].wait()
                g_next = gather(j + 1)
            g.wait()
            st[j] = store(j)
            if j + 1 < nch:
                g = g_next
        if nch >= 2:
            st[nch - 2].wait()
        st[nch - 1].wait()

    return k(table, idx)


_TPW = T // _NW
_NC2 = _TPW // _CH


def _sc_dispatch(x_flat, posd):
    mesh = plsc.VectorSubcoreMesh(core_axis_name="c", subcore_axis_name="s")

    @functools.partial(
        pl.kernel, mesh=mesh,
        out_type=jax.ShapeDtypeStruct((PAD, H), jnp.float32),
        scratch_types=[
            pltpu.VMEM((2, _NC2, _CH), jnp.int32),
            pltpu.VMEM((_CH, H), jnp.float32),
            pltpu.VMEM((_CH, H), jnp.float32),
            pltpu.SemaphoreType.DMA,
            pltpu.SemaphoreType.DMA,
            pltpu.SemaphoreType.DMA,
            pltpu.SemaphoreType.DMA,
        ],
    )
    def k(x_hbm, pos_hbm, xs_hbm, idx_v, buf0, buf1, ra, rb, sa, sb):
        wid = lax.axis_index("s") * _NC + lax.axis_index("c")
        base = wid * _TPW
        pltpu.sync_copy(pos_hbm.at[wid], idx_v)
        bufs = (buf0, buf1)
        rsem = (ra, rb)
        ssem = (sa, sb)

        def read(c):
            return pltpu.async_copy(
                x_hbm.at[pl.ds(base + c * _CH, _CH)], bufs[c % 2], rsem[c % 2])

        def scat(c, slot):
            return pltpu.async_copy(
                bufs[c % 2], xs_hbm.at[idx_v.at[slot, c]], ssem[c % 2])

        rd = read(0)
        wr = {}
        for c in range(_NC2):
            if c + 1 < _NC2:
                if c >= 1:
                    wr[(c - 1, 0)].wait()
                    wr[(c - 1, 1)].wait()
                rd_next = read(c + 1)
            rd.wait()
            wr[(c, 0)] = scat(c, 0)
            wr[(c, 1)] = scat(c, 1)
            if c + 1 < _NC2:
                rd = rd_next
        for c in (_NC2 - 2, _NC2 - 1):
            wr[(c, 0)].wait()
            wr[(c, 1)].wait()

    return k(x_flat, posd)


def _comb_body(g0_ref, g1_ref, w1_ref, w2_ref, o_ref):
    o_ref[...] = w1_ref[...] * g0_ref[...] + w2_ref[...] * g1_ref[...]


def _comb_call(gall, w1, w2):
    row = pl.BlockSpec((TB, H), lambda i: (i, 0))
    row1 = pl.BlockSpec((TB, H), lambda i: (i + NTB, 0))
    col = pl.BlockSpec((TB, 1), lambda i: (i, 0))
    return pl.pallas_call(
        _comb_body,
        grid=(NTB,),
        in_specs=[row, row1, col, col],
        out_specs=row,
        out_shape=jax.ShapeDtypeStruct((T, H), jnp.float32),
    )(gall, gall, w1, w2)


def kernel(x, router, w_up_gate, w_down):
    x_flat = x.reshape(T, H)
    i1, i2, w1, w2, r0, r1, cnt, stats = _router_call(x_flat, router)

    cnt_v = cnt[0]
    cnt_i = cnt_v.astype(jnp.int32)
    tiles_e = (cnt_i + TILE - 1) // TILE
    seg = jnp.concatenate([jnp.zeros(1, jnp.int32),
                           jnp.cumsum(tiles_e * TILE)[:-1]])
    pos0 = seg[i1[:, 0]] + r0[:, 0].astype(jnp.int32)
    pos1 = seg[i2[:, 0]] + r1[:, 0].astype(jnp.int32)
    posd = jnp.stack([pos0.reshape(_NW, _NC2, _CH),
                      pos1.reshape(_NW, _NC2, _CH)], axis=1)

    cumt = jnp.cumsum(tiles_e)
    ti = jnp.arange(NT, dtype=jnp.int32)
    te = jnp.minimum(jnp.sum((ti[:, None] >= cumt[None, :]).astype(jnp.int32),
                             axis=1), E - 1).astype(jnp.int32)
    act = (ti < cumt[-1]).astype(jnp.int32)

    xs = _sc_dispatch(x_flat, posd)
    ys = _mlp_call(te, act, xs, w_up_gate, w_down)
    gall = _sc_gather(ys, jnp.concatenate([pos0, pos1]), A)
    routed = _comb_call(gall, w1, w2).reshape(B, S, H)

    return (routed, cnt_v, stats[0, 0], stats[0, 1], stats[0, 2])


# speedup vs baseline: 1.6867x; 1.0147x over previous
import functools

import jax
import jax.numpy as jnp
from jax import lax
from jax.experimental import pallas as pl
from jax.experimental.pallas import tpu as pltpu
from jax.experimental.pallas import tpu_sc as plsc

H = 1024
I = 2816
E = 8
K = 2
B, S = 2, 2048
T = B * S
A = K * T

TB = 1024
NTB = T // TB

TILE = 512
NT = A // TILE + E
PAD = NT * TILE
NJ = 2
IC = I // NJ


def _router_body(x_ref, r_ref, i1_ref, i2_ref, w1_ref, w2_ref, r0_ref, r1_ref,
                 cnt_ref, stats_ref, carry, psum, zsq):
    bi = pl.program_id(0)

    @pl.when(bi == 0)
    def _init():
        carry[...] = jnp.zeros_like(carry)
        psum[...] = jnp.zeros_like(psum)
        zsq[0, 0] = 0.0

    x = x_ref[...]
    logits = jnp.dot(x, r_ref[...], preferred_element_type=jnp.float32)

    lanes = jax.lax.broadcasted_iota(jnp.int32, (TB, E), 1)
    m1 = jnp.max(logits, axis=-1, keepdims=True)
    i1 = jnp.min(jnp.where(logits == m1, lanes, E), axis=-1, keepdims=True)
    oh1 = (lanes == i1).astype(jnp.float32)
    l2 = jnp.where(lanes == i1, jnp.float32(-1e30), logits)
    m2 = jnp.max(l2, axis=-1, keepdims=True)
    i2 = jnp.min(jnp.where(l2 == m2, lanes, E), axis=-1, keepdims=True)
    oh2 = (lanes == i2).astype(jnp.float32)

    e2 = jnp.exp(m2 - m1)
    w1 = 1.0 / (1.0 + e2)
    w2 = e2 / (1.0 + e2)

    ex = jnp.exp(logits - m1)
    sex = jnp.sum(ex, axis=-1, keepdims=True)
    probs = ex / sex
    z = m1 + jnp.log(sex)

    rows = jax.lax.broadcasted_iota(jnp.int32, (TB, TB), 0)
    cols = jax.lax.broadcasted_iota(jnp.int32, (TB, TB), 1)
    tril = (cols < rows).astype(jnp.float32)
    ohb = oh1 + oh2
    base = jnp.dot(tril, ohb, preferred_element_type=jnp.float32) + carry[...]
    r0 = jnp.sum(base * oh1, axis=-1, keepdims=True)
    r1 = jnp.sum(base * oh2, axis=-1, keepdims=True)

    carry[...] = carry[...] + jnp.sum(ohb, axis=0, keepdims=True)
    psum[...] = psum[...] + jnp.sum(probs, axis=0, keepdims=True)
    zsq[0, 0] = zsq[0, 0] + jnp.sum(z * z)

    i1_ref[...] = i1
    i2_ref[...] = i2
    w1_ref[...] = w1
    w2_ref[...] = w2
    r0_ref[...] = r0
    r1_ref[...] = r1

    @pl.when(bi == NTB - 1)
    def _fin():
        cnt = carry[...]
        cnt_ref[...] = cnt
        af = cnt / jnp.float32(A)
        ent = -jnp.sum(af * jnp.log(af + 1e-6))
        pm = psum[...] / jnp.float32(T)
        lb = jnp.float32(E) * jnp.sum(af * jnp.float32(K) * pm)
        stats_ref[0, 0] = ent
        stats_ref[0, 1] = lb
        stats_ref[0, 2] = zsq[0, 0] / jnp.float32(T)


def _router_call(x_flat, router):
    f32 = jnp.float32
    outs = (
        jax.ShapeDtypeStruct((T, 1), jnp.int32),
        jax.ShapeDtypeStruct((T, 1), jnp.int32),
        jax.ShapeDtypeStruct((T, 1), f32),
        jax.ShapeDtypeStruct((T, 1), f32),
        jax.ShapeDtypeStruct((T, 1), f32),
        jax.ShapeDtypeStruct((T, 1), f32),
        jax.ShapeDtypeStruct((1, E), f32),
        jax.ShapeDtypeStruct((1, 8), f32),
    )
    col = pl.BlockSpec((TB, 1), lambda i: (i, 0))
    return pl.pallas_call(
        _router_body,
        grid=(NTB,),
        in_specs=[
            pl.BlockSpec((TB, H), lambda i: (i, 0)),
            pl.BlockSpec((H, E), lambda i: (0, 0)),
        ],
        out_specs=(col, col, col, col, col, col,
                   pl.BlockSpec((1, E), lambda i: (0, 0)),
                   pl.BlockSpec((1, 8), lambda i: (0, 0),
                                memory_space=pltpu.SMEM)),
        out_shape=outs,
        scratch_shapes=[
            pltpu.VMEM((1, E), f32),
            pltpu.VMEM((1, E), f32),
            pltpu.SMEM((1, 1), f32),
        ],
    )(x_flat, router)


def _mlp_body(te_ref, act_ref, xs_ref, wg_ref, wu_ref, wd_ref, wr_ref,
              out_ref):
    i = pl.program_id(0)
    j = pl.program_id(1)

    @pl.when(act_ref[i] == 1)
    def _():
        xs = xs_ref[...]
        g = jnp.dot(xs, wg_ref[0], preferred_element_type=jnp.float32)
        u = jnp.dot(xs, wu_ref[0], preferred_element_type=jnp.float32)
        hh = g * jax.nn.sigmoid(g) * u
        y = jnp.dot(hh, wd_ref[0], preferred_element_type=jnp.float32)

        @pl.when(j == 0)
        def _():
            out_ref[...] = y

        @pl.when(j != 0)
        def _():
            out_ref[...] = (out_ref[...] + y) * wr_ref[:, 0:1]


def _snake(i, j):
    return jnp.where(i % 2 == 1, NJ - 1 - j, j)


def _mlp_call(te, act, xs, w_up_gate, w_down, w_row):
    grid_spec = pltpu.PrefetchScalarGridSpec(
        num_scalar_prefetch=2,
        grid=(NT, NJ),
        in_specs=[
            pl.BlockSpec((TILE, H), lambda i, j, te, act: (i, 0)),
            pl.BlockSpec((1, H, IC),
                         lambda i, j, te, act: (te[i], 0, _snake(i, j))),
            pl.BlockSpec((1, H, IC),
                         lambda i, j, te, act: (te[i], 0, NJ + _snake(i, j))),
            pl.BlockSpec((1, IC, H),
                         lambda i, j, te, act: (te[i], _snake(i, j), 0)),
            pl.BlockSpec((TILE, 128), lambda i, j, te, act: (i, 0)),
        ],
        out_specs=pl.BlockSpec((TILE, H), lambda i, j, te, act: (i, 0)),
    )
    return pl.pallas_call(
        _mlp_body,
        grid_spec=grid_spec,
        out_shape=jax.ShapeDtypeStruct((PAD, H), jnp.float32),
    )(te, act, xs, w_up_gate, w_up_gate, w_down, w_row)


_NC, _NS = 2, 16
_NW = _NC * _NS
_CH = 32


_TPW = T // _NW
_NC2 = _TPW // _CH


def _sc_dispatch(x_flat, posd, wgt):
    mesh = plsc.VectorSubcoreMesh(core_axis_name="c", subcore_axis_name="s")

    @functools.partial(
        pl.kernel, mesh=mesh,
        out_type=(jax.ShapeDtypeStruct((PAD, H), jnp.float32),
                  jax.ShapeDtypeStruct((PAD, 128), jnp.float32)),
        scratch_types=[
            pltpu.VMEM((2, _NC2, _CH), jnp.int32),
            pltpu.VMEM((2 * _NC2, _CH, 128), jnp.float32),
            pltpu.VMEM((_CH, H), jnp.float32),
            pltpu.VMEM((_CH, H), jnp.float32),
            pltpu.SemaphoreType.DMA,
            pltpu.SemaphoreType.DMA,
            pltpu.SemaphoreType.DMA,
            pltpu.SemaphoreType.DMA,
            pltpu.SemaphoreType.DMA,
        ],
    )
    def k(x_hbm, pos_hbm, w_hbm, xs_hbm, wrow_hbm,
          idx_v, w_v, buf0, buf1, ra, rb, sa, sb, sw):
        wid = lax.axis_index("s") * _NC + lax.axis_index("c")
        base = wid * _TPW
        pltpu.sync_copy(pos_hbm.at[wid], idx_v)
        pltpu.sync_copy(w_hbm.at[wid], w_v)
        bufs = (buf0, buf1)
        rsem = (ra, rb)
        ssem = (sa, sb)

        def read(c):
            return pltpu.async_copy(
                x_hbm.at[pl.ds(base + c * _CH, _CH)], bufs[c % 2], rsem[c % 2])

        def scat(c, slot):
            return pltpu.async_copy(
                bufs[c % 2], xs_hbm.at[idx_v.at[slot, c]], ssem[c % 2])

        wsc = []
        for c in range(_NC2):
            for slot in (0, 1):
                wsc.append(pltpu.async_copy(
                    w_v.at[slot * _NC2 + c], wrow_hbm.at[idx_v.at[slot, c]],
                    sw))

        rd = read(0)
        wr = {}
        for c in range(_NC2):
            if c + 1 < _NC2:
                if c >= 1:
                    wr[(c - 1, 0)].wait()
                    wr[(c - 1, 1)].wait()
                rd_next = read(c + 1)
            rd.wait()
            wr[(c, 0)] = scat(c, 0)
            wr[(c, 1)] = scat(c, 1)
            if c + 1 < _NC2:
                rd = rd_next
        for c in (_NC2 - 2, _NC2 - 1):
            wr[(c, 0)].wait()
            wr[(c, 1)].wait()
        for cp in wsc:
            cp.wait()

    return k(x_flat, posd, wgt)


def _sc_combine(ys, posd):
    mesh = plsc.VectorSubcoreMesh(core_axis_name="c", subcore_axis_name="s")
    cch = 16
    ncc = _TPW // cch
    posc = posd.reshape(_NW, 2, ncc, cch)

    @functools.partial(
        pl.kernel, mesh=mesh,
        out_type=jax.ShapeDtypeStruct((T, H), jnp.float32),
        scratch_types=[
            pltpu.VMEM((2, ncc, cch), jnp.int32),
            pltpu.VMEM((cch, H), jnp.float32),
            pltpu.VMEM((cch, H), jnp.float32),
            pltpu.VMEM((cch, H), jnp.float32),
            pltpu.VMEM((cch, H), jnp.float32),
            pltpu.SemaphoreType.DMA,
            pltpu.SemaphoreType.DMA,
            pltpu.SemaphoreType.DMA,
            pltpu.SemaphoreType.DMA,
            pltpu.SemaphoreType.DMA,
            pltpu.SemaphoreType.DMA,
        ],
    )
    def k(ys_hbm, pos_hbm, out_hbm, idx_v, a0, b0, a1, b1,
          ga0, gb0, ga1, gb1, s0, s1):
        wid = lax.axis_index("s") * _NC + lax.axis_index("c")
        base = wid * _TPW
        pltpu.sync_copy(pos_hbm.at[wid], idx_v)
        abuf = (a0, a1)
        bbuf = (b0, b1)
        gasem = (ga0, ga1)
        gbsem = (gb0, gb1)
        ssem = (s0, s1)

        def gather(c):
            p = c % 2
            return (pltpu.async_copy(ys_hbm.at[idx_v.at[0, c]], abuf[p],
                                     gasem[p]),
                    pltpu.async_copy(ys_hbm.at[idx_v.at[1, c]], bbuf[p],
                                     gbsem[p]))

        def add_rows(c):
            p = c % 2
            a, b = abuf[p], bbuf[p]

            def body(r, _):
                for kk in range(H // 16):
                    sl = pl.ds(kk * 16, 16)
                    a[r, sl] = a[r, sl] + b[r, sl]
                return 0

            lax.fori_loop(0, cch, body, 0)

        def store(c):
            return pltpu.async_copy(
                abuf[c % 2], out_hbm.at[pl.ds(base + c * cch, cch)],
                ssem[c % 2])

        g = gather(0)
        st = {}
        for c in range(ncc):
            if c + 1 < ncc:
                if c >= 1:
                    st[c - 1].wait()
                g_next = gather(c + 1)
            g[0].wait()
            g[1].wait()
            add_rows(c)
            st[c] = store(c)
            if c + 1 < ncc:
                g = g_next
        for c in (ncc - 2, ncc - 1):
            st[c].wait()

    return k(ys, posc)


def kernel(x, router, w_up_gate, w_down):
    x_flat = x.reshape(T, H)
    i1, i2, w1, w2, r0, r1, cnt, stats = _router_call(x_flat, router)

    cnt_v = cnt[0]
    cnt_i = cnt_v.astype(jnp.int32)
    tiles_e = (cnt_i + TILE - 1) // TILE
    seg = jnp.concatenate([jnp.zeros(1, jnp.int32),
                           jnp.cumsum(tiles_e * TILE)[:-1]])
    pos0 = seg[i1[:, 0]] + r0[:, 0].astype(jnp.int32)
    pos1 = seg[i2[:, 0]] + r1[:, 0].astype(jnp.int32)
    posd = jnp.stack([pos0.reshape(_NW, _NC2, _CH),
                      pos1.reshape(_NW, _NC2, _CH)], axis=1)
    wgt = jnp.broadcast_to(
        jnp.stack([w1.reshape(_NW, _NC2, _CH),
                   w2.reshape(_NW, _NC2, _CH)], axis=1)
        .reshape(_NW, 2 * _NC2, _CH, 1), (_NW, 2 * _NC2, _CH, 128))

    cumt = jnp.cumsum(tiles_e)
    ti = jnp.arange(NT, dtype=jnp.int32)
    te = jnp.minimum(jnp.sum((ti[:, None] >= cumt[None, :]).astype(jnp.int32),
                             axis=1), E - 1).astype(jnp.int32)
    act = (ti < cumt[-1]).astype(jnp.int32)

    xs, w_row = _sc_dispatch(x_flat, posd, wgt)
    ys = _mlp_call(te, act, xs, w_up_gate, w_down, w_row)
    routed = _sc_combine(ys, posd).reshape(B, S, H)

    return (routed, cnt_v, stats[0, 0], stats[0, 1], stats[0, 2])
